# baseline (speedup 1.0000x reference)
import functools

import jax
import jax.numpy as jnp
from jax import lax
from jax.experimental import pallas as pl
from jax.experimental.pallas import tpu as pltpu
from jax.experimental.pallas import tpu_sc as plsc

_NC = 2
_NS = 16
_NW = _NC * _NS

_NB = 6


def _make_gather(S, V, Dm, B):
    b_per_w = B // _NW
    nb = min(_NB, S)
    mesh = plsc.VectorSubcoreMesh(core_axis_name="c", subcore_axis_name="s")

    scratch = [pltpu.VMEM((b_per_w,), jnp.int32)]
    scratch += [pltpu.VMEM((b_per_w, Dm), jnp.float32) for _ in range(nb)]
    scratch += [pltpu.SemaphoreType.DMA] * (2 * nb)

    @functools.partial(
        pl.kernel,
        mesh=mesh,
        out_type=jax.ShapeDtypeStruct((S, B, Dm), jnp.float32),
        scratch_types=scratch,
    )
    def gather_kernel(table_hbm, idx_hbm, out_hbm, idx_v, *bufs_and_sems):
        bufs = bufs_and_sems[:nb]
        gsem = bufs_and_sems[nb : 2 * nb]
        ssem = bufs_and_sems[2 * nb :]

        wid = lax.axis_index("s") * _NC + lax.axis_index("c")
        base = wid * b_per_w
        pltpu.sync_copy(idx_hbm.at[pl.ds(base, b_per_w)], idx_v)

        def start_gather(j):
            return pltpu.async_copy(
                table_hbm.at[j].at[idx_v], bufs[j % nb], gsem[j % nb]
            )

        def start_scatter(j):
            return pltpu.async_copy(
                bufs[j % nb], out_hbm.at[j].at[pl.ds(base, b_per_w)], ssem[j % nb]
            )

        gd = [None] * S
        sd = [None] * S
        for j in range(nb):
            gd[j] = start_gather(j)
        for j in range(S):
            if j >= 1 and j - 1 + nb < S:
                sd[j - 1].wait()
                gd[j - 1 + nb] = start_gather(j - 1 + nb)
            gd[j].wait()
            sd[j] = start_scatter(j)
        for j in range(max(0, S - nb), S):
            if sd[j] is not None:
                sd[j].wait()

    return gather_kernel


def kernel(indexs, entity_prompts):
    B = indexs.shape[0]
    V, S, Dm = entity_prompts.shape
    table_t = jnp.transpose(entity_prompts, (1, 0, 2))
    out_t = _make_gather(S, V, Dm, B)(table_t, indexs.astype(jnp.int32))
    return jnp.transpose(out_t, (1, 0, 2))
